# baseline (speedup 1.0000x reference)
import jax
import jax.numpy as jnp
from jax.experimental import pallas as pl
from jax.experimental.pallas import tpu as pltpu

N = 16384
D = 32
K = 5
TOPK = K + 1
BQ = 128
G = N // BQ
C = 128
NV = N // C
NCAND = 8
BIG = 2 * N + 2


def _normalize_kernel(x_ref, o_ref):
    x = x_ref[...]
    n = jnp.sqrt(jnp.sum(x * x, axis=1, keepdims=True))
    o_ref[...] = x / jnp.maximum(n, 1e-12)


def _extract_full(sims, qid, kid):
    match = qid == kid
    iota2 = jax.lax.broadcasted_iota(jnp.int32, (BQ, N), 1) * 2 + 1
    piota = jnp.where(match, iota2 - 1, iota2)
    acc = jnp.zeros((BQ, 1), jnp.int32)
    for k in range(TOPK):
        m = jnp.max(sims, axis=1, keepdims=True)
        key = jnp.min(jnp.where(sims == m, piota, BIG), axis=1,
                      keepdims=True)
        if k > 0:
            acc = acc + (1 - (key & 1))
        if k < TOPK - 1:
            sims = jnp.where(piota == key, -jnp.inf, sims)
    return acc


def _knn_kernel(qid_ref, qn_ref, kid_ref, knt_ref, out_ref):
    qn = qn_ref[...]
    knt = knt_ref[...]
    sims = jax.lax.dot_general(
        qn, knt, (((1,), (0,)), ((), ())),
        preferred_element_type=jnp.float32)

    qid = qid_ref[...]
    kid = kid_ref[...]

    cm = sims[:, 0:C]
    for a in range(1, NV):
        cm = jnp.maximum(cm, sims[:, a * C:(a + 1) * C])

    ciota = jax.lax.broadcasted_iota(jnp.int32, (BQ, C), 1)
    cmw = cm
    cjs = []
    m_last = None
    for _ in range(NCAND):
        m_last = jnp.max(cmw, axis=1, keepdims=True)
        ci = jnp.min(jnp.where(cmw == m_last, ciota, C), axis=1,
                     keepdims=True)
        cjs.append(ci)
        cmw = jnp.where(ciota == ci, -jnp.inf, cmw)
    cjs = jnp.concatenate(cjs, axis=1)
    t9 = jnp.max(cmw, axis=1, keepdims=True)
    needs_full = jnp.any(t9 >= m_last)

    ga, gk, gp = [], [], []
    for a in range(NV):
        sl = sims[:, a * C:(a + 1) * C]
        ga.append(jnp.take_along_axis(sl, cjs, axis=1))
        kl = jnp.broadcast_to(kid[:, a * C:(a + 1) * C], (BQ, C))
        gk.append(jnp.take_along_axis(kl, cjs, axis=1))
        gp.append(cjs + a * C)
    ga = jnp.concatenate(ga, axis=1)
    gk = jnp.concatenate(gk, axis=1)
    gp = jnp.concatenate(gp, axis=1)
    gm = (gk == qid).astype(jnp.int32)
    gpio = gp * 2 + (1 - gm)

    acc = jnp.zeros((BQ, 1), jnp.int32)
    for k in range(TOPK):
        m = jnp.max(ga, axis=1, keepdims=True)
        key = jnp.min(jnp.where(ga == m, gpio, BIG), axis=1,
                      keepdims=True)
        if k > 0:
            acc = acc + (1 - (key & 1))
        if k < TOPK - 1:
            ga = jnp.where(gpio == key, -jnp.inf, ga)

    acc = jax.lax.cond(needs_full,
                       lambda: _extract_full(sims, qid, kid),
                       lambda: acc)
    out_ref[...] = acc.astype(jnp.float32)


def kernel(query_ids, query_embed, key_ids, key_embed):
    norm = pl.pallas_call(
        _normalize_kernel,
        grid=(G,),
        in_specs=[pl.BlockSpec((BQ, D), lambda i: (i, 0))],
        out_specs=pl.BlockSpec((BQ, D), lambda i: (i, 0)),
        out_shape=jax.ShapeDtypeStruct((N, D), jnp.float32),
        compiler_params=pltpu.CompilerParams(
            dimension_semantics=("parallel",)),
    )
    qn = norm(query_embed)
    kn = norm(key_embed)
    knt = kn.T

    counts = pl.pallas_call(
        _knn_kernel,
        grid=(G,),
        in_specs=[
            pl.BlockSpec((BQ, 1), lambda i: (i, 0)),
            pl.BlockSpec((BQ, D), lambda i: (i, 0)),
            pl.BlockSpec((1, N), lambda i: (0, 0)),
            pl.BlockSpec((D, N), lambda i: (0, 0)),
        ],
        out_specs=pl.BlockSpec((BQ, 1), lambda i: (i, 0)),
        out_shape=jax.ShapeDtypeStruct((N, 1), jnp.float32),
        compiler_params=pltpu.CompilerParams(
            dimension_semantics=("parallel",)),
    )(query_ids.reshape(N, 1), qn, key_ids.reshape(1, N), knt)

    return jnp.sum(counts) / jnp.float32(N * K)


# speedup vs baseline: 1.2113x; 1.2113x over previous
import jax
import jax.numpy as jnp
from jax.experimental import pallas as pl
from jax.experimental.pallas import tpu as pltpu

N = 16384
D = 32
K = 5
TOPK = K + 1
BQ = 256
G = N // BQ
C = 128
NV = N // C
NCAND = 8
NW = NV * NCAND
BIGF = float(2 * N + 2)


def _normalize_kernel(x_ref, o_ref):
    x = x_ref[...]
    n = jnp.sqrt(jnp.sum(x * x, axis=1, keepdims=True))
    o_ref[...] = x / jnp.maximum(n, 1e-12)


def _knn_kernel(qid_ref, qn_ref, kid_ref, knt_ref, out_ref, ga_ref, gp_ref):
    qn = qn_ref[...]
    knt = knt_ref[...]
    sims = jax.lax.dot_general(
        qn, knt, (((1,), (0,)), ((), ())),
        preferred_element_type=jnp.float32)

    qid = qid_ref[...]
    kid = kid_ref[...]

    cm = sims[:, 0:C]
    pm = jnp.zeros((BQ, C), jnp.int32)
    for a in range(1, NV):
        sl = sims[:, a * C:(a + 1) * C]
        upd = sl > cm
        pm = jnp.where(upd, a, pm)
        cm = jnp.where(upd, sl, cm)

    ciota = jax.lax.broadcasted_iota(jnp.int32, (BQ, C), 1)
    gidxf = (pm * C + ciota).astype(jnp.float32)
    cmw = cm
    cjs = []
    for _ in range(NCAND):
        m = jnp.max(cmw, axis=1, keepdims=True)
        sel = jnp.min(jnp.where(cmw == m, gidxf, BIGF), axis=1,
                      keepdims=True)
        ci = sel.astype(jnp.int32) & (C - 1)
        cjs.append(ci)
        cmw = jnp.where(ciota == ci, -jnp.inf, cmw)
    cjs = jnp.concatenate(cjs, axis=1)

    for a in range(NV):
        sl = sims[:, a * C:(a + 1) * C]
        gav = jnp.take_along_axis(sl, cjs, axis=1)
        kl = jnp.broadcast_to(kid[:, a * C:(a + 1) * C], (BQ, C))
        gkv = jnp.take_along_axis(kl, cjs, axis=1)
        gpv = (cjs + a * C) * 2 + 1 - (gkv == qid).astype(jnp.int32)
        ga_ref[:, a * NCAND:(a + 1) * NCAND] = gav
        gp_ref[:, a * NCAND:(a + 1) * NCAND] = gpv.astype(jnp.float32)

    ga = ga_ref[...]
    gpio = gp_ref[...]

    acc = jnp.zeros((BQ, 1), jnp.int32)
    for k in range(TOPK):
        m = jnp.max(ga, axis=1, keepdims=True)
        key = jnp.min(jnp.where(ga == m, gpio, BIGF), axis=1,
                      keepdims=True)
        if k > 0:
            acc = acc + (1 - (key.astype(jnp.int32) & 1))
        if k < TOPK - 1:
            ga = jnp.where(gpio == key, -jnp.inf, ga)
    out_ref[...] = acc.astype(jnp.float32)


def kernel(query_ids, query_embed, key_ids, key_embed):
    norm = pl.pallas_call(
        _normalize_kernel,
        grid=(G,),
        in_specs=[pl.BlockSpec((BQ, D), lambda i: (i, 0))],
        out_specs=pl.BlockSpec((BQ, D), lambda i: (i, 0)),
        out_shape=jax.ShapeDtypeStruct((N, D), jnp.float32),
        compiler_params=pltpu.CompilerParams(
            dimension_semantics=("parallel",)),
    )
    qn = norm(query_embed)
    kn = norm(key_embed)
    knt = kn.T

    counts = pl.pallas_call(
        _knn_kernel,
        grid=(G,),
        in_specs=[
            pl.BlockSpec((BQ, 1), lambda i: (i, 0)),
            pl.BlockSpec((BQ, D), lambda i: (i, 0)),
            pl.BlockSpec((1, N), lambda i: (0, 0)),
            pl.BlockSpec((D, N), lambda i: (0, 0)),
        ],
        out_specs=pl.BlockSpec((BQ, 1), lambda i: (i, 0)),
        out_shape=jax.ShapeDtypeStruct((N, 1), jnp.float32),
        scratch_shapes=[
            pltpu.VMEM((BQ, NW), jnp.float32),
            pltpu.VMEM((BQ, NW), jnp.float32),
        ],
        compiler_params=pltpu.CompilerParams(
            dimension_semantics=("parallel",)),
    )(query_ids.reshape(N, 1), qn, key_ids.reshape(1, N), knt)

    return jnp.sum(counts) / jnp.float32(N * K)


# speedup vs baseline: 1.7256x; 1.4246x over previous
import jax
import jax.numpy as jnp
from jax.experimental import pallas as pl
from jax.experimental.pallas import tpu as pltpu

N = 16384
D = 32
K = 5
TOPK = K + 1
BQ = 256
G = N // BQ


def _normalize_kernel(x_ref, o_ref):
    x = x_ref[...]
    n = jnp.sqrt(jnp.sum(x * x, axis=1, keepdims=True))
    o_ref[...] = x / jnp.maximum(n, 1e-12)


def _knn_kernel(qid_ref, qn_ref, kid_ref, knt_ref, out_ref):
    qn = qn_ref[...]
    knt = knt_ref[...]
    sims = jax.lax.dot_general(
        qn, knt, (((1,), (0,)), ((), ())),
        preferred_element_type=jnp.float32)

    qid = qid_ref[...]
    kid = kid_ref[...]
    match = (qid == kid)

    iota2 = jax.lax.broadcasted_iota(jnp.int32, (BQ, N), 1) * 2 + 1
    piota = jnp.where(match, iota2 - 1, iota2).astype(jnp.float32)

    acc = jnp.zeros((BQ, 1), jnp.int32)
    for k in range(TOPK):
        m = jnp.max(sims, axis=1, keepdims=True)
        key = jnp.min(jnp.where(sims == m, piota, float(2 * N)), axis=1,
                      keepdims=True)
        if k > 0:
            acc = acc + (1 - (key.astype(jnp.int32) & 1))
        if k < TOPK - 1:
            sims = jnp.where(piota == key, -jnp.inf, sims)
    out_ref[...] = acc.astype(jnp.float32)


def kernel(query_ids, query_embed, key_ids, key_embed):
    norm = pl.pallas_call(
        _normalize_kernel,
        grid=(G,),
        in_specs=[pl.BlockSpec((BQ, D), lambda i: (i, 0))],
        out_specs=pl.BlockSpec((BQ, D), lambda i: (i, 0)),
        out_shape=jax.ShapeDtypeStruct((N, D), jnp.float32),
        compiler_params=pltpu.CompilerParams(
            dimension_semantics=("parallel",)),
    )
    qn = norm(query_embed)
    kn = norm(key_embed)
    knt = kn.T

    counts = pl.pallas_call(
        _knn_kernel,
        grid=(G,),
        in_specs=[
            pl.BlockSpec((BQ, 1), lambda i: (i, 0)),
            pl.BlockSpec((BQ, D), lambda i: (i, 0)),
            pl.BlockSpec((1, N), lambda i: (0, 0)),
            pl.BlockSpec((D, N), lambda i: (0, 0)),
        ],
        out_specs=pl.BlockSpec((BQ, 1), lambda i: (i, 0)),
        out_shape=jax.ShapeDtypeStruct((N, 1), jnp.float32),
        compiler_params=pltpu.CompilerParams(
            dimension_semantics=("parallel",)),
    )(query_ids.reshape(N, 1), qn, key_ids.reshape(1, N), knt)

    return jnp.sum(counts) / jnp.float32(N * K)
